# baseline (speedup 1.0000x reference)
import functools

import jax
import jax.numpy as jnp
from jax import lax
from jax.experimental import pallas as pl
from jax.experimental.pallas import tpu as pltpu
from jax.experimental.pallas import tpu_sc as plsc

NUM_TABLES = 26
VOCAB = 100000
VOCAB_PAD = 100096
D = 16
BATCH = 4096
L = 20

NC = 2
NS = 16
NW = NC * NS
B_PER_W = BATCH // NW
N_CHUNKS = 2 * NUM_TABLES
BAGS_PER_CHUNK = B_PER_W // 2
ROWS_PER_CHUNK = BAGS_PER_CHUNK * L
INV_L = 1.0 / L

_mesh = plsc.VectorSubcoreMesh(core_axis_name="c", subcore_axis_name="s")


@functools.partial(
    pl.kernel,
    mesh=_mesh,
    compiler_params=pltpu.CompilerParams(use_tc_tiling_on_sc=False),
    out_type=jax.ShapeDtypeStruct((BATCH, NUM_TABLES * D), jnp.float32),
    scratch_types=[
        pltpu.VMEM((2, ROWS_PER_CHUNK), jnp.int32),
        pltpu.VMEM((2, ROWS_PER_CHUNK, D), jnp.float32),
        pltpu.VMEM((B_PER_W, NUM_TABLES * D), jnp.float32),
        pltpu.SemaphoreType.DMA,
        pltpu.SemaphoreType.DMA,
        pltpu.SemaphoreType.DMA,
        pltpu.SemaphoreType.DMA,
    ],
)
def _ebag(idx_hbm, tab_hbm, out_hbm, idx_v, rows_v, ob_v, g0, g1, i0, i1):
    wid = lax.axis_index("s") * NC + lax.axis_index("c")
    b0 = wid * B_PER_W
    gsem = (g0, g1)
    isem = (i0, i1)

    def idx_ofs(c):
        return (c // 2) * (BATCH * L) + b0 * L + (c % 2) * ROWS_PER_CHUNK

    pltpu.sync_copy(idx_hbm.at[pl.ds(idx_ofs(0), ROWS_PER_CHUNK)], idx_v.at[0])
    pltpu.async_copy(tab_hbm.at[idx_v.at[0]], rows_v.at[0], gsem[0])
    pltpu.async_copy(idx_hbm.at[pl.ds(idx_ofs(1), ROWS_PER_CHUNK)],
                     idx_v.at[1], isem[1])

    def two_chunks(cc, _):
        for b in range(2):
            c = cc + b
            o = b ^ 1

            @pl.when(c + 1 < N_CHUNKS)
            def _():
                pltpu.make_async_copy(
                    idx_hbm.at[pl.ds(idx_ofs(c + 1), ROWS_PER_CHUNK)],
                    idx_v.at[o], isem[o]).wait()
                pltpu.async_copy(tab_hbm.at[idx_v.at[o]], rows_v.at[o],
                                 gsem[o])

            pltpu.make_async_copy(tab_hbm.at[idx_v.at[b]], rows_v.at[b],
                                  gsem[b]).wait()

            @pl.when(c + 2 < N_CHUNKS)
            def _():
                pltpu.async_copy(idx_hbm.at[pl.ds(idx_ofs(c + 2),
                                                  ROWS_PER_CHUNK)],
                                 idx_v.at[b], isem[b])

            d0 = (c // 2) * D
            r0 = b * BAGS_PER_CHUNK

            def bag_step(i, _):
                base = i * L
                v = [rows_v[b, base + l, :] for l in range(L)]
                while len(v) > 1:
                    nxt = [v[j] + v[j + 1] for j in range(0, len(v) - 1, 2)]
                    if len(v) % 2:
                        nxt.append(v[-1])
                    v = nxt
                ob_v[r0 + i, pl.ds(d0, D)] = v[0] * INV_L
                return 0

            lax.fori_loop(0, BAGS_PER_CHUNK, bag_step, 0)
        return 0

    lax.fori_loop(0, N_CHUNKS // 2, lambda k, s: two_chunks(2 * k, s), 0)
    pltpu.sync_copy(ob_v, out_hbm.at[pl.ds(b0, B_PER_W)])


V_SPLIT = 17
V_CHUNK = VOCAB_PAD // V_SPLIT
O_CHUNK = V_CHUNK // 8


def _detile_body(tin_ref, out_ref):
    out_ref[...] = tin_ref[0]


def _to_rows(tin):
    return pl.pallas_call(
        _detile_body,
        grid=(NUM_TABLES, V_SPLIT),
        in_specs=[pl.BlockSpec((1, V_CHUNK, D), lambda t, j: (t, j, 0))],
        out_specs=pl.BlockSpec((V_CHUNK, D),
                               lambda t, j: (t * V_SPLIT + j, 0)),
        out_shape=jax.ShapeDtypeStruct((NUM_TABLES * VOCAB_PAD, D),
                                       jnp.float32),
    )(tin)


def kernel(indices, offsets, tables):
    del offsets
    flat = _to_rows(tables)
    adj = indices + (indices // VOCAB) * (VOCAB_PAD - VOCAB)
    return _ebag(adj, flat)


# speedup vs baseline: 2.8797x; 2.8797x over previous
import functools

import jax
import jax.numpy as jnp
from jax import lax
from jax.experimental import pallas as pl
from jax.experimental.pallas import tpu as pltpu
from jax.experimental.pallas import tpu_sc as plsc

NUM_TABLES = 26
VOCAB = 100000
VOCAB_PAD = 100096
D = 16
BATCH = 4096
L = 20

NC = 2
NS = 16
NW = NC * NS
B_PER_W = BATCH // NW
N_CHUNKS = 2 * NUM_TABLES
BAGS_PER_CHUNK = B_PER_W // 2
ROWS_PER_CHUNK = BAGS_PER_CHUNK * L
INV_L = 1.0 / L

_mesh = plsc.VectorSubcoreMesh(core_axis_name="c", subcore_axis_name="s")


@functools.partial(
    pl.kernel,
    mesh=_mesh,
    compiler_params=pltpu.CompilerParams(use_tc_tiling_on_sc=False),
    out_type=jax.ShapeDtypeStruct((BATCH, NUM_TABLES * D), jnp.float32),
    scratch_types=[
        pltpu.VMEM((2, ROWS_PER_CHUNK), jnp.int32),
        pltpu.VMEM((2, ROWS_PER_CHUNK, D), jnp.float32),
        pltpu.VMEM((B_PER_W, NUM_TABLES * D), jnp.float32),
        pltpu.SemaphoreType.DMA,
        pltpu.SemaphoreType.DMA,
        pltpu.SemaphoreType.DMA,
        pltpu.SemaphoreType.DMA,
    ],
)
def _ebag(idx_hbm, tab_hbm, out_hbm, idx_v, rows_v, ob_v, g0, g1, i0, i1):
    wid = lax.axis_index("s") * NC + lax.axis_index("c")
    b0 = wid * B_PER_W
    gsem = (g0, g1)
    isem = (i0, i1)

    def idx_ofs(c):
        return (c // 2) * (BATCH * L) + b0 * L + (c % 2) * ROWS_PER_CHUNK

    pltpu.sync_copy(idx_hbm.at[pl.ds(idx_ofs(0), ROWS_PER_CHUNK)], idx_v.at[0])
    pltpu.async_copy(tab_hbm.at[idx_v.at[0]], rows_v.at[0], gsem[0])
    pltpu.async_copy(idx_hbm.at[pl.ds(idx_ofs(1), ROWS_PER_CHUNK)],
                     idx_v.at[1], isem[1])

    def two_chunks(cc, _):
        for b in range(2):
            c = cc + b
            o = b ^ 1

            @pl.when(c + 1 < N_CHUNKS)
            def _():
                pltpu.make_async_copy(
                    idx_hbm.at[pl.ds(idx_ofs(c + 1), ROWS_PER_CHUNK)],
                    idx_v.at[o], isem[o]).wait()
                pltpu.async_copy(tab_hbm.at[idx_v.at[o]], rows_v.at[o],
                                 gsem[o])

            pltpu.make_async_copy(tab_hbm.at[idx_v.at[b]], rows_v.at[b],
                                  gsem[b]).wait()

            @pl.when(c + 2 < N_CHUNKS)
            def _():
                pltpu.async_copy(idx_hbm.at[pl.ds(idx_ofs(c + 2),
                                                  ROWS_PER_CHUNK)],
                                 idx_v.at[b], isem[b])

            d0 = (c // 2) * D
            r0 = b * BAGS_PER_CHUNK

            def bag_step(i, _):
                base = i * L
                v = [rows_v[b, base + l, :] for l in range(L)]
                while len(v) > 1:
                    nxt = [v[j] + v[j + 1] for j in range(0, len(v) - 1, 2)]
                    if len(v) % 2:
                        nxt.append(v[-1])
                    v = nxt
                ob_v[r0 + i, pl.ds(d0, D)] = v[0] * INV_L
                return 0

            lax.fori_loop(0, BAGS_PER_CHUNK, bag_step, 0)
        return 0

    lax.fori_loop(0, N_CHUNKS // 2, lambda k, s: two_chunks(2 * k, s), 0)
    pltpu.sync_copy(ob_v, out_hbm.at[pl.ds(b0, B_PER_W)])


V_SPLIT = 17
V_CHUNK = VOCAB_PAD // V_SPLIT
O_CHUNK = V_CHUNK // 8


def _detile_body(tin_ref, out_ref):
    x = tin_ref[0]
    y = jnp.transpose(x)
    y4 = y.reshape(O_CHUNK // 8, 8, 8, D) + 0.0
    out_ref[...] = y4.reshape(O_CHUNK, 8 * D)


def _to_rows(tin):
    return pl.pallas_call(
        _detile_body,
        grid=(NUM_TABLES, V_SPLIT),
        in_specs=[pl.BlockSpec((1, D, V_CHUNK), lambda t, j: (t, 0, j))],
        out_specs=pl.BlockSpec((O_CHUNK, 8 * D),
                               lambda t, j: (t * V_SPLIT + j, 0)),
        out_shape=jax.ShapeDtypeStruct((NUM_TABLES * VOCAB_PAD // 8, 8 * D),
                                       jnp.float32),
    )(tin)


def kernel(indices, offsets, tables):
    del offsets
    tin = jnp.swapaxes(tables, 1, 2)
    flat = _to_rows(tin).reshape(NUM_TABLES * VOCAB_PAD, D)
    adj = indices + (indices // VOCAB) * (VOCAB_PAD - VOCAB)
    return _ebag(adj, flat)


# speedup vs baseline: 3.0193x; 1.0485x over previous
import functools

import jax
import jax.numpy as jnp
from jax import lax
from jax.experimental import pallas as pl
from jax.experimental.pallas import tpu as pltpu
from jax.experimental.pallas import tpu_sc as plsc

NUM_TABLES = 26
VOCAB = 100000
VOCAB_PAD = 100096
D = 16
BATCH = 4096
L = 20

NC = 2
NS = 16
NW = NC * NS
B_PER_W = BATCH // NW
N_CHUNKS = 2 * NUM_TABLES
BAGS_PER_CHUNK = B_PER_W // 2
ROWS_PER_CHUNK = BAGS_PER_CHUNK * L
INV_L = 1.0 / L

_mesh = plsc.VectorSubcoreMesh(core_axis_name="c", subcore_axis_name="s")


def _make_ebag(nt):
    n_chunks = 2 * nt

    @functools.partial(
        pl.kernel,
        mesh=_mesh,
        compiler_params=pltpu.CompilerParams(use_tc_tiling_on_sc=False),
        out_type=jax.ShapeDtypeStruct((BATCH, nt * D), jnp.float32),
        scratch_types=[
            pltpu.VMEM((2, ROWS_PER_CHUNK), jnp.int32),
            pltpu.VMEM((2, ROWS_PER_CHUNK, D), jnp.float32),
            pltpu.VMEM((B_PER_W, nt * D), jnp.float32),
            pltpu.SemaphoreType.DMA,
            pltpu.SemaphoreType.DMA,
            pltpu.SemaphoreType.DMA,
            pltpu.SemaphoreType.DMA,
        ],
    )
    def _ebag(idx_hbm, tab_hbm, out_hbm, idx_v, rows_v, ob_v, g0, g1, i0, i1):
        wid = lax.axis_index("s") * NC + lax.axis_index("c")
        b0 = wid * B_PER_W
        gsem = (g0, g1)
        isem = (i0, i1)

        def idx_ofs(c):
            return (c // 2) * (BATCH * L) + b0 * L + (c % 2) * ROWS_PER_CHUNK

        pltpu.sync_copy(idx_hbm.at[pl.ds(idx_ofs(0), ROWS_PER_CHUNK)],
                        idx_v.at[0])
        pltpu.async_copy(tab_hbm.at[idx_v.at[0]], rows_v.at[0], gsem[0])
        pltpu.async_copy(idx_hbm.at[pl.ds(idx_ofs(1), ROWS_PER_CHUNK)],
                         idx_v.at[1], isem[1])

        def two_chunks(cc, _):
            for b in range(2):
                c = cc + b
                o = b ^ 1

                @pl.when(c + 1 < n_chunks)
                def _():
                    pltpu.make_async_copy(
                        idx_hbm.at[pl.ds(idx_ofs(c + 1), ROWS_PER_CHUNK)],
                        idx_v.at[o], isem[o]).wait()
                    pltpu.async_copy(tab_hbm.at[idx_v.at[o]], rows_v.at[o],
                                     gsem[o])

                pltpu.make_async_copy(tab_hbm.at[idx_v.at[b]], rows_v.at[b],
                                      gsem[b]).wait()

                @pl.when(c + 2 < n_chunks)
                def _():
                    pltpu.async_copy(idx_hbm.at[pl.ds(idx_ofs(c + 2),
                                                      ROWS_PER_CHUNK)],
                                     idx_v.at[b], isem[b])

                d0 = (c // 2) * D
                r0 = b * BAGS_PER_CHUNK

                def bag_step(i, _):
                    base = i * L
                    v = [rows_v[b, base + l, :] for l in range(L)]
                    while len(v) > 1:
                        nxt = [v[j] + v[j + 1]
                               for j in range(0, len(v) - 1, 2)]
                        if len(v) % 2:
                            nxt.append(v[-1])
                        v = nxt
                    ob_v[r0 + i, pl.ds(d0, D)] = v[0] * INV_L
                    return 0

                lax.fori_loop(0, BAGS_PER_CHUNK, bag_step, 0)
            return 0

        lax.fori_loop(0, n_chunks // 2, lambda k, s: two_chunks(2 * k, s), 0)
        pltpu.sync_copy(ob_v, out_hbm.at[pl.ds(b0, B_PER_W)])

    return _ebag


NT_A = 13
NT_B = NUM_TABLES - NT_A
_ebag_a = _make_ebag(NT_A)
_ebag_b = _make_ebag(NT_B)


V_SPLIT = 17
V_CHUNK = VOCAB_PAD // V_SPLIT
O_CHUNK = V_CHUNK // 8


def _detile_body(tin_ref, out_ref):
    x = tin_ref[0]
    y = jnp.transpose(x)
    y4 = y.reshape(O_CHUNK // 8, 8, 8, D) + 0.0
    out_ref[...] = y4.reshape(O_CHUNK, 8 * D)


def _to_rows(tin, t0, nt):
    return pl.pallas_call(
        _detile_body,
        grid=(nt, V_SPLIT),
        in_specs=[pl.BlockSpec((1, D, V_CHUNK), lambda t, j: (t0 + t, 0, j))],
        out_specs=pl.BlockSpec((O_CHUNK, 8 * D),
                               lambda t, j: (t * V_SPLIT + j, 0)),
        out_shape=jax.ShapeDtypeStruct((nt * VOCAB_PAD // 8, 8 * D),
                                       jnp.float32),
    )(tin)


def kernel(indices, offsets, tables):
    del offsets
    tin = jnp.swapaxes(tables, 1, 2)
    adj = indices + (indices // VOCAB) * (VOCAB_PAD - VOCAB)
    n_a = NT_A * BATCH * L
    adj_a = adj[:n_a]
    adj_b = adj[n_a:] - NT_A * VOCAB_PAD
    flat_a = _to_rows(tin, 0, NT_A).reshape(NT_A * VOCAB_PAD, D)
    out_a = _ebag_a(adj_a, flat_a)
    flat_b = _to_rows(tin, NT_A, NT_B).reshape(NT_B * VOCAB_PAD, D)
    out_b = _ebag_b(adj_b, flat_b)
    return jnp.concatenate([out_a, out_b], axis=1)
